# Initial kernel scaffold; baseline (speedup 1.0000x reference)
import jax
import jax.numpy as jnp
from jax.experimental import pallas as pl


def kernel(inputs, edges, W, a):
    raise NotImplementedError("write your pallas kernel here")


# speedup vs baseline: 8.4483x; 8.4483x over previous
import functools

import jax
import jax.numpy as jnp
from jax import lax
from jax.experimental import pallas as pl
from jax.experimental.pallas import tpu as pltpu
from jax.experimental.pallas import tpu_sc as plsc

K_EDGE = 512
R_SPAN = 512
GC = 128


def _mm_body(x_ref, w_ref, a2_ref, h_ref, s_ref):
    h = jnp.dot(x_ref[...], w_ref[...], preferred_element_type=jnp.float32)
    h_ref[...] = h
    s_ref[...] = jnp.dot(h, a2_ref[...], preferred_element_type=jnp.float32)


def _transform(x, W, A2, row_blk):
    N, F = x.shape
    U = W.shape[1]
    grid = (N // row_blk,)
    return pl.pallas_call(
        _mm_body,
        grid=grid,
        in_specs=[
            pl.BlockSpec((row_blk, F), lambda b: (b, 0)),
            pl.BlockSpec((F, U), lambda b: (0, 0)),
            pl.BlockSpec((U, 2), lambda b: (0, 0)),
        ],
        out_specs=[
            pl.BlockSpec((row_blk, U), lambda b: (b, 0)),
            pl.BlockSpec((row_blk, 2), lambda b: (b, 0)),
        ],
        out_shape=[
            jax.ShapeDtypeStruct((N, U), jnp.float32),
            jax.ShapeDtypeStruct((N, 2), jnp.float32),
        ],
    )(x, W, A2)


def _make_sc_edges(N, U, E, E_pad):
    info = plsc.get_sparse_core_info()
    NC, NS, L = info.num_cores, info.num_subcores, info.num_lanes
    NW = NC * NS
    EC = E_pad // NW
    n_full = EC // GC
    rem = EC % GC

    mesh = plsc.VectorSubcoreMesh(core_axis_name="c", subcore_axis_name="s")

    @functools.partial(
        pl.kernel,
        mesh=mesh,
        compiler_params=pltpu.CompilerParams(needs_layout_passes=False),
        out_type=[
            jax.ShapeDtypeStruct((E_pad,), jnp.float32),
            jax.ShapeDtypeStruct((E_pad, U), jnp.float32),
        ],
        scratch_types=[
            pltpu.VMEM((N,), jnp.float32),
            pltpu.VMEM((N,), jnp.float32),
            pltpu.VMEM((EC,), jnp.int32),
            pltpu.VMEM((EC,), jnp.int32),
            pltpu.VMEM((EC,), jnp.float32),
            pltpu.VMEM((GC, U), jnp.float32),
            pltpu.SemaphoreType.DMA,
        ],
    )
    def sc_edges(s_hbm, src_hbm, dst_hbm, h_hbm, score_hbm, g_hbm,
                 s1_v, s2_v, src_v, dst_v, score_v, rows_v, sem):
        wid = lax.axis_index("s") * NC + lax.axis_index("c")
        base = wid * EC
        pltpu.sync_copy(s_hbm.at[0], s1_v)
        pltpu.sync_copy(s_hbm.at[1], s2_v)
        pltpu.sync_copy(src_hbm.at[pl.ds(base, EC)], src_v)
        pltpu.sync_copy(dst_hbm.at[pl.ds(base, EC)], dst_v)

        def score_body(i, carry):
            o = i * L
            sv = src_v[pl.ds(o, L)]
            dv = dst_v[pl.ds(o, L)]
            t = plsc.load_gather(s1_v, [sv]) + plsc.load_gather(s2_v, [dv])
            t = jnp.maximum(t, 0.2 * t)
            t = jnp.clip(t, -2.0, 2.0)
            sc = jnp.exp(t)
            gid = base + o + lax.iota(jnp.int32, L)
            sc = jnp.where(gid < E, sc, 0.0)
            score_v[pl.ds(o, L)] = sc
            return carry

        lax.fori_loop(0, EC // L, score_body, 0)
        pltpu.sync_copy(score_v, score_hbm.at[pl.ds(base, EC)])

        def gather_body(k, carry):
            off = k * GC
            pltpu.async_copy(h_hbm.at[dst_v.at[pl.ds(off, GC)]], rows_v,
                             sem).wait()
            pltpu.sync_copy(rows_v, g_hbm.at[pl.ds(base + off, GC)])
            return carry

        lax.fori_loop(0, n_full, gather_body, 0)
        if rem:
            off = n_full * GC
            pltpu.async_copy(h_hbm.at[dst_v.at[pl.ds(off, rem)]],
                             rows_v.at[pl.ds(0, rem)], sem).wait()
            pltpu.sync_copy(rows_v.at[pl.ds(0, rem)],
                            g_hbm.at[pl.ds(base + off, rem)])

    return sc_edges


def _agg_body(base_sref, g_ref, sc_ref, src_ref, acc_ref, sums_ref, *, nb):
    b = pl.program_id(0)

    @pl.when(b == 0)
    def _init():
        acc_ref[...] = jnp.zeros_like(acc_ref)
        sums_ref[...] = jnp.zeros_like(sums_ref)

    base = pl.multiple_of((base_sref[b] // 8) * 8, 8)
    loc = src_ref[0] - base
    iot = lax.broadcasted_iota(jnp.int32, (R_SPAN, K_EDGE), 0)
    S = jnp.where(iot == loc, sc_ref[0], 0.0)
    contrib = jnp.dot(S, g_ref[...], preferred_element_type=jnp.float32)
    rsum = jnp.sum(S, axis=1, keepdims=True)
    acc_ref[pl.ds(base, R_SPAN), :] += contrib
    sums_ref[pl.ds(base, R_SPAN), :] += rsum

    @pl.when(b == nb - 1)
    def _fin():
        sv = sums_ref[...]
        acc_ref[...] = acc_ref[...] / jnp.where(sv > 0.0, sv, 1.0)


def _aggregate(base_arr, G, score3, src3, N, U):
    nb = score3.shape[0]
    NA = N + R_SPAN
    grid_spec = pltpu.PrefetchScalarGridSpec(
        num_scalar_prefetch=1,
        grid=(nb,),
        in_specs=[
            pl.BlockSpec((K_EDGE, U), lambda b, s: (b, 0)),
            pl.BlockSpec((1, 1, K_EDGE), lambda b, s: (b, 0, 0)),
            pl.BlockSpec((1, 1, K_EDGE), lambda b, s: (b, 0, 0)),
        ],
        out_specs=[
            pl.BlockSpec((NA, U), lambda b, s: (0, 0)),
            pl.BlockSpec((NA, 1), lambda b, s: (0, 0)),
        ],
    )
    acc, _ = pl.pallas_call(
        functools.partial(_agg_body, nb=nb),
        grid_spec=grid_spec,
        out_shape=[
            jax.ShapeDtypeStruct((NA, U), jnp.float32),
            jax.ShapeDtypeStruct((NA, 1), jnp.float32),
        ],
    )(base_arr, G, score3, src3)
    return acc[:N]


def kernel(inputs, edges, W, a):
    B, N, F = inputs.shape
    U = W.shape[1]
    E = edges.shape[0]
    nb = -(-E // K_EDGE)
    E_pad = nb * K_EDGE

    x = inputs.reshape(N, F)
    A2 = jnp.concatenate([a[:U], a[U:]], axis=1)

    h, s = _transform(x, W, A2, row_blk=1000)
    sT = s.T

    src = edges[:, 0]
    dst = edges[:, 1]
    pad = E_pad - E
    src_p = jnp.concatenate([src, jnp.full((pad,), N - 1, jnp.int32)])
    dst_p = jnp.concatenate([dst, jnp.zeros((pad,), jnp.int32)])

    sc_edges = _make_sc_edges(N, U, E, E_pad)
    score, G = sc_edges(sT, src_p, dst_p, h)

    base_arr = src_p[0::K_EDGE]
    score3 = score.reshape(nb, 1, K_EDGE)
    src3 = src_p.reshape(nb, 1, K_EDGE)

    out = _aggregate(base_arr, G, score3, src3, N, U)
    return out.reshape(B, N, U)


# speedup vs baseline: 10.6961x; 1.2661x over previous
import functools

import jax
import jax.numpy as jnp
from jax import lax
from jax.experimental import pallas as pl
from jax.experimental.pallas import tpu as pltpu
from jax.experimental.pallas import tpu_sc as plsc

K_EDGE = 1024
R_SPAN = 256
GC = 128


def _mm_body(x_ref, w_ref, a2_ref, h_ref, s_ref):
    h = jnp.dot(x_ref[...], w_ref[...], preferred_element_type=jnp.float32)
    h_ref[...] = h
    s_ref[...] = jnp.dot(h, a2_ref[...], preferred_element_type=jnp.float32)


def _transform(x, W, A2, row_blk):
    N, F = x.shape
    U = W.shape[1]
    grid = (N // row_blk,)
    return pl.pallas_call(
        _mm_body,
        grid=grid,
        in_specs=[
            pl.BlockSpec((row_blk, F), lambda b: (b, 0)),
            pl.BlockSpec((F, U), lambda b: (0, 0)),
            pl.BlockSpec((U, 2), lambda b: (0, 0)),
        ],
        out_specs=[
            pl.BlockSpec((row_blk, U), lambda b: (b, 0)),
            pl.BlockSpec((row_blk, 2), lambda b: (b, 0)),
        ],
        out_shape=[
            jax.ShapeDtypeStruct((N, U), jnp.float32),
            jax.ShapeDtypeStruct((N, 2), jnp.float32),
        ],
    )(x, W, A2)


def _make_sc_edges(N, U, E, E_pad):
    info = plsc.get_sparse_core_info()
    NC, NS, L = info.num_cores, info.num_subcores, info.num_lanes
    NW = NC * NS
    EC = E_pad // NW
    n_full = EC // GC
    rem = EC % GC

    mesh = plsc.VectorSubcoreMesh(core_axis_name="c", subcore_axis_name="s")

    @functools.partial(
        pl.kernel,
        mesh=mesh,
        compiler_params=pltpu.CompilerParams(needs_layout_passes=False),
        out_type=[
            jax.ShapeDtypeStruct((E_pad,), jnp.float32),
            jax.ShapeDtypeStruct((E_pad, U), jnp.float32),
        ],
        scratch_types=[
            pltpu.VMEM((N,), jnp.float32),
            pltpu.VMEM((N,), jnp.float32),
            pltpu.VMEM((EC,), jnp.int32),
            pltpu.VMEM((EC,), jnp.int32),
            pltpu.VMEM((EC,), jnp.float32),
            pltpu.VMEM((2, GC, U), jnp.float32),
            pltpu.SemaphoreType.DMA((2,)),
        ],
    )
    def sc_edges(s_hbm, src_hbm, dst_hbm, h_hbm, score_hbm, g_hbm,
                 s1_v, s2_v, src_v, dst_v, score_v, rows_v, sem):
        wid = lax.axis_index("s") * NC + lax.axis_index("c")
        base = wid * EC
        pltpu.sync_copy(s_hbm.at[0], s1_v)
        pltpu.sync_copy(s_hbm.at[1], s2_v)
        pltpu.sync_copy(src_hbm.at[pl.ds(base, EC)], src_v)
        pltpu.sync_copy(dst_hbm.at[pl.ds(base, EC)], dst_v)

        def score_body(i, carry):
            o = i * L
            sv = src_v[pl.ds(o, L)]
            dv = dst_v[pl.ds(o, L)]
            t = plsc.load_gather(s1_v, [sv]) + plsc.load_gather(s2_v, [dv])
            t = jnp.maximum(t, 0.2 * t)
            t = jnp.clip(t, -2.0, 2.0)
            sc = jnp.exp(t)
            gid = base + o + lax.iota(jnp.int32, L)
            sc = jnp.where(gid < E, sc, 0.0)
            score_v[pl.ds(o, L)] = sc
            return carry

        lax.fori_loop(0, EC // L, score_body, 0)
        pltpu.sync_copy(score_v, score_hbm.at[pl.ds(base, EC)])

        def _start(k, b):
            pltpu.async_copy(h_hbm.at[dst_v.at[pl.ds(k * GC, GC)]],
                             rows_v.at[b], sem.at[b])

        def _wait(k, b):
            pltpu.make_async_copy(h_hbm.at[dst_v.at[pl.ds(k * GC, GC)]],
                                  rows_v.at[b], sem.at[b]).wait()

        _start(0, 0)
        if n_full > 1:
            _start(1, 1)

        def pair_body(p, carry):
            k0 = p * 2
            for b in range(2):
                k = k0 + b
                _wait(k, b)
                pltpu.sync_copy(rows_v.at[b],
                                g_hbm.at[pl.ds(base + k * GC, GC)])
                nk = k + 2

                @pl.when(nk < n_full)
                def _():
                    _start(nk, b)
            return carry

        lax.fori_loop(0, n_full // 2, pair_body, 0)
        if n_full % 2:
            k = n_full - 1
            _wait(k, 0)
            pltpu.sync_copy(rows_v.at[0], g_hbm.at[pl.ds(base + k * GC, GC)])
        if rem:
            off = n_full * GC
            pltpu.async_copy(h_hbm.at[dst_v.at[pl.ds(off, rem)]],
                             rows_v.at[0, pl.ds(0, rem)], sem.at[0]).wait()
            pltpu.sync_copy(rows_v.at[0, pl.ds(0, rem)],
                            g_hbm.at[pl.ds(base + off, rem)])

    return sc_edges


def _agg_body(base_sref, g_ref, sc_ref, src_ref, acc_ref, sums_ref, *, nb):
    b = pl.program_id(0)

    @pl.when(b == 0)
    def _init():
        acc_ref[...] = jnp.zeros_like(acc_ref)
        sums_ref[...] = jnp.zeros_like(sums_ref)

    base = pl.multiple_of((base_sref[b] // 8) * 8, 8)
    loc = src_ref[0] - base
    iot = lax.broadcasted_iota(jnp.int32, (R_SPAN, K_EDGE), 0)
    S = jnp.where(iot == loc, sc_ref[0], 0.0)
    contrib = jnp.dot(S, g_ref[...], preferred_element_type=jnp.float32)
    rsum = jnp.sum(S, axis=1, keepdims=True)
    acc_ref[pl.ds(base, R_SPAN), :] += contrib
    sums_ref[pl.ds(base, R_SPAN), :] += rsum

    @pl.when(b == nb - 1)
    def _fin():
        sv = sums_ref[...]
        acc_ref[...] = acc_ref[...] / jnp.where(sv > 0.0, sv, 1.0)


def _aggregate(base_arr, G, score3, src3, N, U):
    nb = score3.shape[0]
    NA = N + R_SPAN
    grid_spec = pltpu.PrefetchScalarGridSpec(
        num_scalar_prefetch=1,
        grid=(nb,),
        in_specs=[
            pl.BlockSpec((K_EDGE, U), lambda b, s: (b, 0)),
            pl.BlockSpec((1, 1, K_EDGE), lambda b, s: (b, 0, 0)),
            pl.BlockSpec((1, 1, K_EDGE), lambda b, s: (b, 0, 0)),
        ],
        out_specs=[
            pl.BlockSpec((NA, U), lambda b, s: (0, 0)),
            pl.BlockSpec((NA, 1), lambda b, s: (0, 0)),
        ],
    )
    acc, _ = pl.pallas_call(
        functools.partial(_agg_body, nb=nb),
        grid_spec=grid_spec,
        out_shape=[
            jax.ShapeDtypeStruct((NA, U), jnp.float32),
            jax.ShapeDtypeStruct((NA, 1), jnp.float32),
        ],
    )(base_arr, G, score3, src3)
    return acc[:N]


def kernel(inputs, edges, W, a):
    B, N, F = inputs.shape
    U = W.shape[1]
    E = edges.shape[0]
    nb = -(-E // K_EDGE)
    E_pad = nb * K_EDGE

    x = inputs.reshape(N, F)
    A2 = jnp.concatenate([a[:U], a[U:]], axis=1)

    h, s = _transform(x, W, A2, row_blk=1000)
    sT = s.T

    src = edges[:, 0]
    dst = edges[:, 1]
    pad = E_pad - E
    src_p = jnp.concatenate([src, jnp.full((pad,), N - 1, jnp.int32)])
    dst_p = jnp.concatenate([dst, jnp.zeros((pad,), jnp.int32)])

    sc_edges = _make_sc_edges(N, U, E, E_pad)
    score, G = sc_edges(sT, src_p, dst_p, h)

    base_arr = src_p[0::K_EDGE]
    score3 = score.reshape(nb, 1, K_EDGE)
    src3 = src_p.reshape(nb, 1, K_EDGE)

    out = _aggregate(base_arr, G, score3, src3, N, U)
    return out.reshape(B, N, U)
